# baseline (speedup 1.0000x reference)
import functools

import jax
import jax.numpy as jnp
from jax import lax
from jax.experimental import pallas as pl
from jax.experimental.pallas import tpu as pltpu
from jax.experimental.pallas import tpu_sc as plsc

B = 16384
G = 20
D = 64
LANES = 16
NC = 2
NS = 16
NW = NC * NS
ROWS_PER_W = B // NW
CHUNK = 32
NCHUNK = ROWS_PER_W // CHUNK
IDX_PER_CHUNK = CHUNK * G
ELEM_PER_CHUNK = IDX_PER_CHUNK * D

BUCKET = 2000000
TILE_MINOR = 128
TILE_MAJOR = 8
N_TC = BUCKET // TILE_MINOR
FJ_STRIDE = N_TC * TILE_MAJOR * TILE_MINOR


def _ngram_body(ngram_hbm, wsub_hbm, out_hbm,
                nidx_v, eidx_v, elem_v, out_v, sem):
    wid = lax.axis_index("s") * NC + lax.axis_index("c")
    base_row = wid * ROWS_PER_W

    lane = lax.iota(jnp.int32, LANES)
    offs = []
    for d in range(D // LANES):
        j = lane + d * LANES
        offs.append(((j >> 3) * FJ_STRIDE) + ((j & 7) << 7))

    def chunk_body(i, carry):
        row0 = base_row + i * CHUNK
        pltpu.sync_copy(ngram_hbm.at[pl.ds(row0 * G, IDX_PER_CHUNK)],
                        nidx_v.at[pl.ds(0, IDX_PER_CHUNK)])

        def id_body(k, c2):
            nvec = nidx_v[pl.ds(k, LANES)]
            bases = ((nvec >> 7) << 10) | (nvec & (TILE_MINOR - 1))
            for l in range(LANES):
                kd = (k + l) * D
                for d in range(D // LANES):
                    eidx_v[pl.ds(kd + d * LANES, LANES)] = bases[l] + offs[d]
            return c2

        nsp = 4
        rows_per_sp = CHUNK // nsp
        sl_len = ELEM_PER_CHUNK // nsp
        ids_per_sp = IDX_PER_CHUNK // nsp
        gcopies = []
        for s in range(nsp):
            lax.fori_loop(s * ids_per_sp // LANES,
                          (s + 1) * ids_per_sp // LANES,
                          lambda m, c: id_body(m * LANES, c), 0)
            gcopies.append(pltpu.async_copy(
                wsub_hbm.at[eidx_v.at[pl.ds(s * sl_len, sl_len)]],
                elem_v.at[pl.ds(s * sl_len, sl_len)], sem))

        def row_body(r, c2):
            rg = r * G * D
            accs = [elem_v[pl.ds(rg + d * LANES, LANES)]
                    for d in range(D // LANES)]
            for g in range(1, G):
                gd = rg + g * D
                for d in range(D // LANES):
                    accs[d] = accs[d] + elem_v[pl.ds(gd + d * LANES, LANES)]
            for d in range(D // LANES):
                out_v[r, pl.ds(d * LANES, LANES)] = accs[d]
            return c2

        for s in range(nsp):
            gcopies[s].wait()
            lax.fori_loop(s * rows_per_sp, (s + 1) * rows_per_sp, row_body, 0)
        pltpu.sync_copy(out_v, out_hbm.at[pl.ds(row0, CHUNK)])
        return carry

    lax.fori_loop(0, NCHUNK, chunk_body, 0)


def _center_body(center_hbm, win_hbm, part_hbm, out_hbm,
                 cidx_v, crow_v, part_v, sem, psem):
    wid = lax.axis_index("s") * NC + lax.axis_index("c")
    base_row = wid * ROWS_PER_W

    def chunk_body(i, carry):
        row0 = base_row + i * CHUNK
        pltpu.sync_copy(center_hbm.at[pl.ds(row0, CHUNK)], cidx_v)
        gcopy = pltpu.async_copy(win_hbm.at[cidx_v], crow_v, sem)
        pcopy = pltpu.async_copy(part_hbm.at[pl.ds(row0, CHUNK)], part_v, psem)
        gcopy.wait()
        pcopy.wait()

        def row_body(r, c2):
            for d in range(D // LANES):
                sl = pl.ds(d * LANES, LANES)
                part_v[r, sl] = part_v[r, sl] + crow_v[r, sl]
            return c2


        lax.fori_loop(0, CHUNK, row_body, 0)
        pltpu.sync_copy(part_v, out_hbm.at[pl.ds(row0, CHUNK)])
        return carry

    lax.fori_loop(0, NCHUNK, chunk_body, 0)


def kernel(center_ids, ngram_ids, W_in, W_sub):
    center_ids = center_ids.astype(jnp.int32)
    ngram_flat = ngram_ids.astype(jnp.int32).reshape(B * G)
    wsub_flat = (W_sub
                 .reshape(N_TC, TILE_MINOR, D // TILE_MAJOR, TILE_MAJOR)
                 .transpose(2, 0, 3, 1)
                 .reshape(BUCKET * D))
    mesh = plsc.VectorSubcoreMesh(core_axis_name="c", subcore_axis_name="s")
    params = pltpu.CompilerParams(use_tc_tiling_on_sc=False)

    ngram_fn = functools.partial(
        pl.kernel,
        mesh=mesh,
        compiler_params=params,
        out_type=jax.ShapeDtypeStruct((B, D), jnp.float32),
        scratch_types=[
            pltpu.VMEM((IDX_PER_CHUNK + LANES,), jnp.int32),
            pltpu.VMEM((ELEM_PER_CHUNK,), jnp.int32),
            pltpu.VMEM((ELEM_PER_CHUNK,), jnp.float32),
            pltpu.VMEM((CHUNK, D), jnp.float32),
            pltpu.SemaphoreType.DMA,
        ],
    )(_ngram_body)
    partial_sums = ngram_fn(ngram_flat, wsub_flat)

    center_fn = functools.partial(
        pl.kernel,
        mesh=mesh,
        compiler_params=params,
        out_type=jax.ShapeDtypeStruct((B, D), jnp.float32),
        scratch_types=[
            pltpu.VMEM((CHUNK,), jnp.int32),
            pltpu.VMEM((CHUNK, D), jnp.float32),
            pltpu.VMEM((CHUNK, D), jnp.float32),
            pltpu.SemaphoreType.DMA,
            pltpu.SemaphoreType.DMA,
        ],
    )(_center_body)
    return center_fn(center_ids, W_in, partial_sums)


# speedup vs baseline: 1.0037x; 1.0037x over previous
import functools

import jax
import jax.numpy as jnp
from jax import lax
from jax.experimental import pallas as pl
from jax.experimental.pallas import tpu as pltpu
from jax.experimental.pallas import tpu_sc as plsc

B = 16384
G = 20
D = 64
LANES = 16
NC = 2
NS = 16
NW = NC * NS
ROWS_PER_W = B // NW
CHUNK = 32
NCHUNK = ROWS_PER_W // CHUNK
IDX_PER_CHUNK = CHUNK * G
ELEM_PER_CHUNK = IDX_PER_CHUNK * D

BUCKET = 2000000
TILE_MINOR = 128
TILE_MAJOR = 8
N_TC = BUCKET // TILE_MINOR
FJ_STRIDE = N_TC * TILE_MAJOR * TILE_MINOR


def _ngram_body(ngram_hbm, wsub_hbm, out_hbm,
                nidx_v, eidx_v, elem_v, out_v, sem):
    wid = lax.axis_index("s") * NC + lax.axis_index("c")
    base_row = wid * ROWS_PER_W

    lane = lax.iota(jnp.int32, LANES)
    offs = []
    for d in range(D // LANES):
        j = lane + d * LANES
        offs.append(((j >> 3) * FJ_STRIDE) + ((j & 7) << 7))

    pltpu.sync_copy(ngram_hbm.at[pl.ds(base_row * G, ROWS_PER_W * G)],
                    nidx_v.at[pl.ds(0, ROWS_PER_W * G)])

    def chunk_body(i, carry):
        row0 = base_row + i * CHUNK
        idx0 = i * IDX_PER_CHUNK

        def id_body(k, c2):
            nvec = nidx_v[pl.ds(idx0 + k, LANES)]
            bases = ((nvec >> 7) << 10) | (nvec & (TILE_MINOR - 1))
            for l in range(LANES):
                kd = (k + l) * D
                for d in range(D // LANES):
                    eidx_v[pl.ds(kd + d * LANES, LANES)] = bases[l] + offs[d]
            return c2

        nsp = 8
        rows_per_sp = CHUNK // nsp
        sl_len = ELEM_PER_CHUNK // nsp
        ids_per_sp = IDX_PER_CHUNK // nsp
        gcopies = []
        for s in range(nsp):
            lax.fori_loop(s * ids_per_sp // LANES,
                          (s + 1) * ids_per_sp // LANES,
                          lambda m, c: id_body(m * LANES, c), 0)
            gcopies.append(pltpu.async_copy(
                wsub_hbm.at[eidx_v.at[pl.ds(s * sl_len, sl_len)]],
                elem_v.at[pl.ds(s * sl_len, sl_len)], sem))

        def row_body(r, c2):
            rg = r * G * D
            accs = [elem_v[pl.ds(rg + d * LANES, LANES)]
                    for d in range(D // LANES)]
            for g in range(1, G):
                gd = rg + g * D
                for d in range(D // LANES):
                    accs[d] = accs[d] + elem_v[pl.ds(gd + d * LANES, LANES)]
            for d in range(D // LANES):
                out_v[r, pl.ds(d * LANES, LANES)] = accs[d]
            return c2

        for s in range(nsp):
            gcopies[s].wait()
            lax.fori_loop(s * rows_per_sp, (s + 1) * rows_per_sp, row_body, 0)
        pltpu.sync_copy(out_v, out_hbm.at[pl.ds(row0, CHUNK)])
        return carry

    lax.fori_loop(0, NCHUNK, chunk_body, 0)


def _center_body(center_hbm, win_hbm, part_hbm, out_hbm,
                 cidx_v, crow_v, part_v, sem, psem):
    wid = lax.axis_index("s") * NC + lax.axis_index("c")
    base_row = wid * ROWS_PER_W

    def chunk_body(i, carry):
        row0 = base_row + i * CHUNK
        pltpu.sync_copy(center_hbm.at[pl.ds(row0, CHUNK)], cidx_v)
        gcopy = pltpu.async_copy(win_hbm.at[cidx_v], crow_v, sem)
        pcopy = pltpu.async_copy(part_hbm.at[pl.ds(row0, CHUNK)], part_v, psem)
        gcopy.wait()
        pcopy.wait()

        def row_body(r, c2):
            for d in range(D // LANES):
                sl = pl.ds(d * LANES, LANES)
                part_v[r, sl] = part_v[r, sl] + crow_v[r, sl]
            return c2


        lax.fori_loop(0, CHUNK, row_body, 0)
        pltpu.sync_copy(part_v, out_hbm.at[pl.ds(row0, CHUNK)])
        return carry

    lax.fori_loop(0, NCHUNK, chunk_body, 0)


def kernel(center_ids, ngram_ids, W_in, W_sub):
    center_ids = center_ids.astype(jnp.int32)
    ngram_flat = ngram_ids.astype(jnp.int32).reshape(B * G)
    wsub_flat = (W_sub
                 .reshape(N_TC, TILE_MINOR, D // TILE_MAJOR, TILE_MAJOR)
                 .transpose(2, 0, 3, 1)
                 .reshape(BUCKET * D))
    mesh = plsc.VectorSubcoreMesh(core_axis_name="c", subcore_axis_name="s")
    params = pltpu.CompilerParams(use_tc_tiling_on_sc=False)

    ngram_fn = functools.partial(
        pl.kernel,
        mesh=mesh,
        compiler_params=params,
        out_type=jax.ShapeDtypeStruct((B, D), jnp.float32),
        scratch_types=[
            pltpu.VMEM((ROWS_PER_W * G + LANES,), jnp.int32),
            pltpu.VMEM((ELEM_PER_CHUNK,), jnp.int32),
            pltpu.VMEM((ELEM_PER_CHUNK,), jnp.float32),
            pltpu.VMEM((CHUNK, D), jnp.float32),
            pltpu.SemaphoreType.DMA,
        ],
    )(_ngram_body)
    partial_sums = ngram_fn(ngram_flat, wsub_flat)

    center_fn = functools.partial(
        pl.kernel,
        mesh=mesh,
        compiler_params=params,
        out_type=jax.ShapeDtypeStruct((B, D), jnp.float32),
        scratch_types=[
            pltpu.VMEM((CHUNK,), jnp.int32),
            pltpu.VMEM((CHUNK, D), jnp.float32),
            pltpu.VMEM((CHUNK, D), jnp.float32),
            pltpu.SemaphoreType.DMA,
            pltpu.SemaphoreType.DMA,
        ],
    )(_center_body)
    return center_fn(center_ids, W_in, partial_sums)
